# Initial kernel scaffold; baseline (speedup 1.0000x reference)
import jax
import jax.numpy as jnp
from jax.experimental import pallas as pl


def kernel(input_concepts, edge_index, edge_type, global_table, W1, root1, bias1, W2, root2, bias2):
    raise NotImplementedError("write your pallas kernel here")


# speedup vs baseline: 8.4201x; 8.4201x over previous
import functools

import jax
import jax.numpy as jnp
from jax import lax
from jax.experimental import pallas as pl
from jax.experimental.pallas import tpu as pltpu
from jax.experimental.pallas import tpu_sc as plsc

N = 10000
NP = 10240
D = 256
R = 8
E = 160000
K = 10
KP = 16
RNP = R * NP
NEG = -1e30

BM = 256
BM2 = 1024
NB2 = NP // BM2
ECH = E // 128


def _topk_body(x_ref, g_ref, w_ref, i_ref, s_ref):
    x = x_ref[...]
    xn = jnp.sqrt(jnp.sum(x * x, axis=1, keepdims=True))
    xh = x / jnp.maximum(xn, 1e-12)
    g = g_ref[...]
    gn = jnp.sqrt(jnp.sum(g * g, axis=1, keepdims=True))
    gh = g / jnp.maximum(gn, 1e-12)
    s = lax.dot_general(xh, gh, (((1,), (1,)), ((), ())),
                        preferred_element_type=jnp.float32)
    col = lax.broadcasted_iota(jnp.int32, (BM, NP), 1)
    s = jnp.where(col < N, s, NEG)
    s_ref[...] = s
    m = jnp.max(s, axis=1)
    vals, idxs = [], []
    for k in range(K):
        s = s_ref[...]
        j = jnp.min(jnp.where(s == m[:, None], col, NP), axis=1)
        vals.append(m[:, None])
        idxs.append(j[:, None])
        if k < K - 1:
            masked = jnp.where(col == j[:, None], NEG, s)
            m = jnp.max(masked, axis=1)
            s_ref[...] = masked
    v = jnp.concatenate(vals, axis=1)
    ji = jnp.concatenate(idxs, axis=1).astype(jnp.int32)
    e = jnp.exp(v - v[:, 0:1])
    w = e / jnp.sum(e, axis=1, keepdims=True)
    w_ref[...] = jnp.concatenate(
        [w, jnp.zeros((BM, KP - K), jnp.float32)], axis=1)
    i_ref[...] = jnp.concatenate(
        [ji, jnp.zeros((BM, KP - K), jnp.int32)], axis=1)


def _topk(x_pad, g_pad):
    return pl.pallas_call(
        _topk_body,
        grid=(NP // BM,),
        in_specs=[
            pl.BlockSpec((BM, D), lambda i: (i, 0)),
            pl.BlockSpec((NP, D), lambda i: (0, 0)),
        ],
        out_specs=[
            pl.BlockSpec((BM, KP), lambda i: (i, 0)),
            pl.BlockSpec((BM, KP), lambda i: (i, 0)),
        ],
        out_shape=[
            jax.ShapeDtypeStruct((NP, KP), jnp.float32),
            jax.ShapeDtypeStruct((NP, KP), jnp.int32),
        ],
        scratch_shapes=[pltpu.VMEM((BM, NP), jnp.float32)],
    )(x_pad, g_pad)


_MESH = plsc.VectorSubcoreMesh(core_axis_name="c", subcore_axis_name="s")
_IOTA16 = lambda: lax.broadcasted_iota(jnp.int32, (16,), 0)


@functools.partial(
    pl.kernel,
    out_type=jax.ShapeDtypeStruct((2, NP, 128), jnp.float32),
    mesh=_MESH,
    compiler_params=pltpu.CompilerParams(needs_layout_passes=False),
    scratch_types=[
        pltpu.VMEM((128,), jnp.int32),
        pltpu.VMEM((128,), jnp.int32),
        pltpu.VMEM((128, 128), jnp.float32),
        pltpu.VMEM_SHARED((NP, 128), jnp.float32),
    ],
)
def _counts_kernel(dst_hbm, et_hbm, z_hbm, out_hbm, dv, ev, obuf, cnt_sp):
    c = lax.axis_index("c")
    s = lax.axis_index("s")
    w = s * 2 + c
    pltpu.sync_copy(z_hbm, obuf)

    def zblock(i, _):
        pltpu.sync_copy(z_hbm, cnt_sp.at[pl.ds((s * 5 + i) * 128, 128)])
        return 0
    lax.fori_loop(0, 5, zblock, 0)
    plsc.subcore_barrier()

    ones = jnp.full((16,), 1.0, jnp.float32)
    zeros = jnp.zeros((16,), jnp.float32)

    def chunk(i, _):
        ch = w + i * 32
        @pl.when(ch < ECH)
        def _():
            base = ch * 128
            pltpu.sync_copy(dst_hbm.at[pl.ds(base, 128)], dv)
            pltpu.sync_copy(et_hbm.at[pl.ds(base, 128)], ev)
            for q in range(8):
                rows = _IOTA16() + q * 16
                plsc.store_scatter(obuf, [rows, ev[pl.ds(q * 16, 16)]], ones)
            pltpu.sync_copy(obuf, cnt_sp.at[dv], add=True)
            for q in range(8):
                rows = _IOTA16() + q * 16
                plsc.store_scatter(obuf, [rows, ev[pl.ds(q * 16, 16)]], zeros)
        return 0
    lax.fori_loop(0, 40, chunk, 0)
    plsc.subcore_barrier()

    def oblock(i, _):
        r0 = (s * 5 + i) * 128
        pltpu.sync_copy(cnt_sp.at[pl.ds(r0, 128)], out_hbm.at[c, pl.ds(r0, 128)])
        return 0
    lax.fori_loop(0, 5, oblock, 0)


def _inv_body(c_ref, o_ref):
    cboth = c_ref[...]
    o_ref[...] = 1.0 / jnp.maximum(cboth[0] + cboth[1], 1.0)


def _invcnt(cnts):
    return pl.pallas_call(
        _inv_body,
        grid=(NP // 1024,),
        in_specs=[pl.BlockSpec((2, 1024, 128), lambda i: (0, i, 0))],
        out_specs=pl.BlockSpec((1024, 128), lambda i: (i, 0)),
        out_shape=jax.ShapeDtypeStruct((NP, 128), jnp.float32),
    )(cnts)


CB = 8
CG = CB * K
NBLK = NP // CB


@functools.partial(
    pl.kernel,
    out_type=jax.ShapeDtypeStruct((NP, D), jnp.float32),
    mesh=_MESH,
    compiler_params=pltpu.CompilerParams(needs_layout_passes=False),
    scratch_types=[
        pltpu.VMEM((2, CG), jnp.int32),
        pltpu.VMEM((2, CG, D), jnp.float32),
        pltpu.VMEM((2, CG, KP), jnp.float32),
        pltpu.VMEM((CB, D), jnp.float32),
        pltpu.VMEM((CB, D), jnp.float32),
        pltpu.SemaphoreType.DMA,
        pltpu.SemaphoreType.DMA,
    ],
)
def _context_kernel(x_hbm, g_hbm, widx_hbm, wbc_hbm, out_hbm,
                    idx_v, rows_v, wv, xin_v, o_v, sem_a, sem_b):
    w = lax.axis_index("s") * 2 + lax.axis_index("c")
    sems = (sem_a, sem_b)

    def load_side(bi, blk):
        s10 = blk * CG
        pltpu.sync_copy(widx_hbm.at[pl.ds(s10, CG)], idx_v.at[bi])
        pltpu.sync_copy(wbc_hbm.at[pl.ds(s10, CG)], wv.at[bi])
        pltpu.async_copy(g_hbm.at[idx_v.at[bi]], rows_v.at[bi], sems[bi])

    def wait_side(bi):
        pltpu.make_async_copy(g_hbm.at[idx_v.at[bi]],
                              rows_v.at[bi], sems[bi]).wait()

    def compute_side(bi, blk):
        pltpu.sync_copy(x_hbm.at[pl.ds(blk * CB, CB)], xin_v)
        for j in range(CB):
            wl = [wv[bi, j * K + k, :] for k in range(K)]

            def cbody(cc, _):
                off = pl.multiple_of(cc * 16, 16)
                s0 = xin_v[j, pl.ds(off, 16)]
                s1 = wl[0] * rows_v[bi, j * K, pl.ds(off, 16)]
                for k in range(1, K):
                    t = wl[k] * rows_v[bi, j * K + k, pl.ds(off, 16)]
                    if k % 2:
                        s1 = s1 + t
                    else:
                        s0 = s0 + t
                o_v[j, pl.ds(off, 16)] = s0 + s1
                return 0
            lax.fori_loop(0, D // 16, cbody, 0)
        pltpu.sync_copy(o_v, out_hbm.at[pl.ds(blk * CB, CB)])

    load_side(0, w)

    def pair(i, _):
        blk0 = w + (2 * i) * 32
        blk1 = blk0 + 32
        load_side(1, blk1)
        wait_side(0)
        compute_side(0, blk0)

        @pl.when(i < (NBLK // 64) - 1)
        def _():
            load_side(0, blk0 + 64)
        wait_side(1)
        compute_side(1, blk1)
        return 0
    lax.fori_loop(0, NBLK // 64, pair, 0)


def _h_body(x_ref, w_ref, h_ref):
    h_ref[...] = jnp.dot(x_ref[...], w_ref[0],
                         preferred_element_type=jnp.float32)


def _hmats(x, W):
    return pl.pallas_call(
        _h_body,
        grid=(NB2, R, 2),
        in_specs=[
            pl.BlockSpec((BM2, D), lambda n, r, hh: (n, 0)),
            pl.BlockSpec((1, D, 128), lambda n, r, hh: (r, 0, hh)),
        ],
        out_specs=pl.BlockSpec((BM2, 128),
                               lambda n, r, hh: ((hh * R + r) * NB2 + n, 0)),
        out_shape=jax.ShapeDtypeStruct((2 * RNP, 128), jnp.float32),
    )(x, W)


@functools.partial(
    pl.kernel,
    out_type=jax.ShapeDtypeStruct((ECH, 128), jnp.float32),
    mesh=_MESH,
    compiler_params=pltpu.CompilerParams(needs_layout_passes=False),
    scratch_types=[
        pltpu.VMEM((128,), jnp.int32),
        pltpu.VMEM((128,), jnp.int32),
        pltpu.VMEM((1, 128), jnp.float32),
        pltpu.VMEM((128, 128), jnp.float32),
        pltpu.SemaphoreType.DMA,
    ],
)
def _scale_kernel(dst_hbm, et_hbm, inv_hbm, out_hbm, dv, ev, srow, irows, sem1):
    w = lax.axis_index("s") * 2 + lax.axis_index("c")

    def chunk(i, _):
        ch = w + i * 32
        @pl.when(ch < ECH)
        def _():
            base = ch * 128
            pltpu.sync_copy(dst_hbm.at[pl.ds(base, 128)], dv)
            pltpu.sync_copy(et_hbm.at[pl.ds(base, 128)], ev)
            pltpu.async_copy(inv_hbm.at[dv], irows, sem1).wait()
            for q in range(8):
                sl = pl.ds(q * 16, 16)
                srow[0, sl] = plsc.load_gather(
                    irows, [_IOTA16() + q * 16, ev[sl]])
            pltpu.sync_copy(srow, out_hbm.at[pl.ds(ch, 1)])
        return 0
    lax.fori_loop(0, 40, chunk, 0)


@functools.partial(
    pl.kernel,
    out_type=jax.ShapeDtypeStruct((2, NP, 128), jnp.float32),
    mesh=_MESH,
    compiler_params=pltpu.CompilerParams(needs_layout_passes=False),
    scratch_types=[
        pltpu.VMEM((2, 128), jnp.int32),
        pltpu.VMEM((2, 128), jnp.int32),
        pltpu.VMEM((2, 128), jnp.int32),
        pltpu.VMEM((2, 128), jnp.int32),
        pltpu.VMEM((2, 1, 128), jnp.float32),
        pltpu.VMEM((2, 128, 128), jnp.float32),
        pltpu.VMEM_SHARED((NP, 128), jnp.float32),
        pltpu.SemaphoreType.DMA,
        pltpu.SemaphoreType.DMA,
    ],
)
def _agg_kernel(src_hbm, dst_hbm, et_hbm, h_hbm, scale_hbm, z_hbm, out_hbm,
                sv, dv, ev, gidx, srow, hrows, acc_sp, sem_a, sem_b):
    c = lax.axis_index("c")
    s = lax.axis_index("s")
    sems = (sem_a, sem_b)

    def zblock(i, _):
        pltpu.sync_copy(z_hbm, acc_sp.at[pl.ds((s * 5 + i) * 128, 128)])
        return 0
    lax.fori_loop(0, 5, zblock, 0)
    plsc.subcore_barrier()

    half = c * RNP
    zeros16 = jnp.zeros((16,), jnp.int32)

    def load_side(bi, ch):
        base = ch * 128
        pltpu.sync_copy(src_hbm.at[pl.ds(base, 128)], sv.at[bi])
        pltpu.sync_copy(et_hbm.at[pl.ds(base, 128)], ev.at[bi])
        pltpu.sync_copy(dst_hbm.at[pl.ds(base, 128)], dv.at[bi])
        pltpu.sync_copy(scale_hbm.at[pl.ds(ch, 1)], srow.at[bi])
        for q in range(8):
            sl = pl.ds(q * 16, 16)
            gidx[bi, sl] = half + ev[bi, sl] * NP + sv[bi, sl]
        pltpu.async_copy(h_hbm.at[gidx.at[bi]], hrows.at[bi], sems[bi])

    def process_side(bi):
        pltpu.make_async_copy(h_hbm.at[gidx.at[bi]],
                              hrows.at[bi], sems[bi]).wait()
        b16 = jnp.full((16,), bi, jnp.int32)

        @plsc.parallel_loop(0, 128, unroll=2)
        def _(e):
            f16 = jnp.full((16,), e, jnp.int32)
            sc16 = plsc.load_gather(srow, [b16, zeros16, f16])
            for q in range(8):
                slq = pl.ds(q * 16, 16)
                hrows[bi, e, slq] = hrows[bi, e, slq] * sc16
        pltpu.sync_copy(hrows.at[bi], acc_sp.at[dv.at[bi]], add=True)

    load_side(0, s)

    def pair(i, _):
        ch0 = s + (2 * i) * 16
        ch1 = ch0 + 16

        @pl.when(ch1 < ECH)
        def _():
            load_side(1, ch1)

        @pl.when(ch0 < ECH)
        def _():
            process_side(0)

        @pl.when(ch0 + 32 < ECH)
        def _():
            load_side(0, ch0 + 32)

        @pl.when(ch1 < ECH)
        def _():
            process_side(1)
        return 0
    lax.fori_loop(0, 40, pair, 0)
    plsc.subcore_barrier()

    def oblock(i, _):
        r0 = (s * 5 + i) * 128
        pltpu.sync_copy(acc_sp.at[pl.ds(r0, 128)], out_hbm.at[c, pl.ds(r0, 128)])
        return 0
    lax.fori_loop(0, 5, oblock, 0)


def _fuse_body(x_ref, rt_ref, b_ref, a_ref, o_ref, *, do_relu):
    z = jnp.dot(x_ref[...], rt_ref[...], preferred_element_type=jnp.float32)
    a = a_ref[...]
    z = z + b_ref[...] + jnp.concatenate([a[0], a[1]], axis=1)
    o_ref[...] = jnp.maximum(z, 0.0) if do_relu else z


def _fuse_skip_body(x_ref, rt_ref, b_ref, a_ref, skip_ref, o_ref):
    z = jnp.dot(x_ref[...], rt_ref[...], preferred_element_type=jnp.float32)
    a = a_ref[...]
    o_ref[...] = skip_ref[...] + z + b_ref[...] + jnp.concatenate(
        [a[0], a[1]], axis=1)


def _fuse(x, root, bias2d, agg, skip=None):
    in_specs = [
        pl.BlockSpec((BM2, D), lambda n: (n, 0)),
        pl.BlockSpec((D, D), lambda n: (0, 0)),
        pl.BlockSpec((1, D), lambda n: (0, 0)),
        pl.BlockSpec((2, BM2, 128), lambda n: (0, n, 0)),
    ]
    args = [x, root, bias2d, agg]
    if skip is None:
        body = functools.partial(_fuse_body, do_relu=True)
    else:
        body = _fuse_skip_body
        in_specs.append(pl.BlockSpec((BM2, D), lambda n: (n, 0)))
        args.append(skip)
    return pl.pallas_call(
        body,
        grid=(NB2,),
        in_specs=in_specs,
        out_specs=pl.BlockSpec((BM2, D), lambda n: (n, 0)),
        out_shape=jax.ShapeDtypeStruct((NP, D), jnp.float32),
    )(*args)


def kernel(input_concepts, edge_index, edge_type, global_table,
           W1, root1, bias1, W2, root2, bias2):
    f32 = jnp.float32
    x = input_concepts.astype(f32)
    g = global_table.astype(f32)
    src = edge_index[0].astype(jnp.int32)
    dst = edge_index[1].astype(jnp.int32)
    et = edge_type.astype(jnp.int32)

    x_pad = jnp.pad(x, ((0, NP - N), (0, 0)))
    g_pad = jnp.pad(g, ((0, NP - N), (0, 0)))

    wts, idxs = _topk(x_pad, g_pad)

    widx_flat = idxs[:, :K].reshape(-1)
    wbc = jnp.broadcast_to(wts[:, :K, None], (NP, K, KP)).reshape(-1, KP)

    zrows128 = jnp.zeros((128, 128), f32)

    cnts = _counts_kernel(dst, et, zrows128)
    invp = _invcnt(cnts)
    scales = _scale_kernel(dst, et, invp)

    ri = _context_kernel(x_pad, g, widx_flat, wbc)

    b1 = bias1.astype(f32).reshape(1, D)
    b2 = bias2.astype(f32).reshape(1, D)

    h1 = _hmats(ri, W1.astype(f32))
    agg1 = _agg_kernel(src, dst, et, h1, scales, zrows128)
    x1 = _fuse(ri, root1.astype(f32), b1, agg1)

    h2 = _hmats(x1, W2.astype(f32))
    agg2 = _agg_kernel(src, dst, et, h2, scales, zrows128)
    out = _fuse(x1, root2.astype(f32), b2, agg2, skip=ri)

    return out[:N]
